# baseline (speedup 1.0000x reference)
import functools

import jax
import jax.numpy as jnp
from jax import lax
from jax.experimental import pallas as pl
from jax.experimental.pallas import tpu as pltpu
from jax.experimental.pallas import tpu_sc as plsc

_N_CORES = 2
_N_SUBCORES = 16
_NW = _N_CORES * _N_SUBCORES
_BATCH = 16384
_DIM = 64
_B_PER_W = _BATCH // _NW
_CHUNK = 128
_N_CHUNKS = _B_PER_W // _CHUNK


def _gather_kernel(params_hbm, idx_hbm, out_hbm, idx_v, rows_v, sem):
    wid = lax.axis_index("s") * _N_CORES + lax.axis_index("c")
    base = wid * _B_PER_W
    pltpu.sync_copy(idx_hbm.at[wid], idx_v)
    copies = []
    for j in range(_N_CHUNKS):
        copies.append(
            pltpu.async_copy(
                params_hbm.at[idx_v.at[j]],
                rows_v.at[pl.ds(j * _CHUNK, _CHUNK)],
                sem,
            )
        )
    for c in copies:
        c.wait()
    pltpu.sync_copy(rows_v, out_hbm.at[pl.ds(base, _B_PER_W)])


@jax.jit
def kernel(x, params):
    idx = x.reshape(_NW, _N_CHUNKS, _CHUNK)
    run = pl.kernel(
        _gather_kernel,
        out_type=jax.ShapeDtypeStruct((_BATCH, _DIM), jnp.float32),
        mesh=plsc.VectorSubcoreMesh(core_axis_name="c", subcore_axis_name="s"),
        scratch_types=[
            pltpu.VMEM((_N_CHUNKS, _CHUNK), jnp.int32),
            pltpu.VMEM((_B_PER_W, _DIM), jnp.float32),
            pltpu.SemaphoreType.DMA,
        ],
        compiler_params=pltpu.CompilerParams(use_tc_tiling_on_sc=False),
    )
    return run(params, idx)


# speedup vs baseline: 1.6325x; 1.6325x over previous
import jax
import jax.numpy as jnp
from jax import lax
from jax.experimental import pallas as pl
from jax.experimental.pallas import tpu as pltpu
from jax.experimental.pallas import tpu_sc as plsc

_NW = 32
_BATCH = 16384
_DIM = 64
_B_PER_W = _BATCH // _NW
_W = 16


def _body(params_hbm, idx_hbm, out_hbm, idx_s, rows_v, sem):
    wid = lax.axis_index("s") * 2 + lax.axis_index("c")
    base = wid * _B_PER_W
    pltpu.sync_copy(idx_hbm.at[wid], idx_s)

    def window(w, _):
        k0 = w * _W
        vec = idx_s[pl.ds(k0, _W)]
        for j in range(_W):
            s = vec[j]
            pltpu.async_copy(
                params_hbm.at[pl.ds(s, 1), :],
                rows_v.at[pl.ds(k0 + j, 1), :],
                sem,
            )
        for j in range(_W):
            pltpu.make_async_copy(
                params_hbm.at[pl.ds(0, 1), :],
                rows_v.at[pl.ds(k0 + j, 1), :],
                sem,
            ).wait()
        return 0

    lax.fori_loop(0, _B_PER_W // _W, window, 0)
    pltpu.sync_copy(rows_v, out_hbm.at[pl.ds(base, _B_PER_W), :])


@jax.jit
def kernel(x, params):
    idx = x.reshape(_NW, _B_PER_W)
    run = pl.kernel(
        _body,
        out_type=jax.ShapeDtypeStruct((_BATCH, _DIM), jnp.float32),
        mesh=plsc.VectorSubcoreMesh(core_axis_name="c", subcore_axis_name="s"),
        scratch_types=[
            pltpu.VMEM((_B_PER_W,), jnp.int32),
            pltpu.VMEM((_B_PER_W, _DIM), jnp.float32),
            pltpu.SemaphoreType.DMA,
        ],
        compiler_params=pltpu.CompilerParams(use_tc_tiling_on_sc=True),
    )
    return run(params, idx)
